# Initial kernel scaffold; baseline (speedup 1.0000x reference)
import jax
import jax.numpy as jnp
from jax.experimental import pallas as pl


def kernel(vid, Wq, bq, Wk, bk, Wv, bv, Wp, bp):
    raise NotImplementedError("write your pallas kernel here")


# speedup vs baseline: 16.4014x; 16.4014x over previous
import functools

import numpy as np
import jax
import jax.numpy as jnp
from jax import lax
from jax.experimental import pallas as pl
from jax.experimental.pallas import tpu as pltpu
from jax.experimental.pallas import tpu_sc as plsc

EMBED = 32
HEADS = 4
DIM = 128
PS = 5
PS2 = PS * PS
KNN = 10
STRIDE = 4
PADW = 2
T = 2
H = 64
W = 64
N = T * H * W
NG = 16
Q = T * NG * NG
HP = H + 2 * PADW
VROWS = HEADS * T * HP * HP
ITEMS = HEADS * Q
QB = 128
NQB = Q // QB
NC, NS = 2, 16
NW = NC * NS
PER_W = ITEMS // NW
IDXW = 256

_OFF = (np.arange(PS)[:, None] * HP + np.arange(PS)[None, :]).reshape(-1)
_S256 = np.zeros((KNN, IDXW), np.float32)
for _k in range(KNN):
    _S256[_k, _k * PS2:(_k + 1) * PS2] = 1.0
_OFFROW = np.zeros((1, IDXW), np.float32)
_OFFROW[0, :KNN * PS2] = np.tile(_OFF.astype(np.float32), KNN)
_E160 = np.zeros((KNN, KNN * 16), np.float32)
for _k in range(KNN):
    _E160[_k, _k * 16:(_k + 1) * 16] = 1.0

_c1 = np.zeros(HP, np.float32)
for _m in range(NG):
    for _oi in range(PS):
        _c1[_m * STRIDE + _oi] += 1.0
_c1 = _c1[PADW:PADW + H]
_CNT = _c1[:, None] * _c1[None, :]
_CINV = (1.0 / np.maximum(_CNT, 1.0)).astype(np.float32)
_CMASK = (_CNT > 0).astype(np.float32)


def _proj_body(x_ref, wk_ref, bk_ref, wv_ref, bv_ref, k_ref, v_ref):
    xb = x_ref[...]
    dn = (((1,), (1,)), ((), ()))
    k_ref[...] = lax.dot_general(xb, wk_ref[...], dn,
                                 preferred_element_type=jnp.float32) + bk_ref[...]
    v_ref[...] = lax.dot_general(xb, wv_ref[...], dn,
                                 preferred_element_type=jnp.float32) + bv_ref[...]


def _run_proj(x, Wk, bk, Wv, bv):
    nblk = N // 512
    return pl.pallas_call(
        _proj_body,
        grid=(nblk,),
        in_specs=[
            pl.BlockSpec((512, DIM), lambda i: (i, 0)),
            pl.BlockSpec((DIM, DIM), lambda i: (0, 0)),
            pl.BlockSpec((1, DIM), lambda i: (0, 0)),
            pl.BlockSpec((DIM, DIM), lambda i: (0, 0)),
            pl.BlockSpec((1, DIM), lambda i: (0, 0)),
        ],
        out_specs=[
            pl.BlockSpec((512, DIM), lambda i: (i, 0)),
            pl.BlockSpec((512, DIM), lambda i: (i, 0)),
        ],
        out_shape=[
            jax.ShapeDtypeStruct((N, DIM), jnp.float32),
            jax.ShapeDtypeStruct((N, DIM), jnp.float32),
        ],
    )(x, Wk, bk.reshape(1, DIM), Wv, bv.reshape(1, DIM))


def _search_body(xq_ref, wq_ref, bq_ref, kh_ref, s256_ref, offr_ref, e160_ref,
                 idx_ref, probs_ref):
    h = pl.program_id(0)
    dn = (((1,), (1,)), ((), ()))
    xb = xq_ref[...]
    qh = lax.dot_general(xb, wq_ref[0], dn,
                         preferred_element_type=jnp.float32) + bq_ref[0]
    kh = kh_ref[0]
    kn = jnp.sum(kh * kh, axis=1)
    qn = jnp.sum(qh * qh, axis=1)
    qk = lax.dot_general(qh, kh, dn, preferred_element_type=jnp.float32)
    s = (2.0 * qk - qn[:, None]) - kn[None, :]
    iota = lax.broadcasted_iota(jnp.int32, (QB, N), 1)
    vals, inds = [], []
    for _ in range(KNN):
        m = jnp.max(s, axis=1, keepdims=True)
        a = jnp.argmax(s, axis=1).astype(jnp.int32)[:, None]
        vals.append(m)
        inds.append(a)
        s = jnp.where(iota == a, -jnp.inf, s)
    v10 = jnp.concatenate(vals, axis=1)
    i10 = jnp.concatenate(inds, axis=1)
    e = jnp.exp(v10 - v10[:, 0:1])
    p = e / jnp.sum(e, axis=1, keepdims=True)
    probs_ref[...] = lax.dot_general(p, e160_ref[...], (((1,), (0,)), ((), ())),
                                     preferred_element_type=jnp.float32, precision=lax.Precision.HIGHEST)
    t = i10 >> 12
    rem = i10 & 4095
    y = rem >> 6
    xw = rem & 63
    base = ((h * T + t) * HP + y) * HP + xw
    basef = base.astype(jnp.float32)
    idxf = lax.dot_general(basef, s256_ref[...], (((1,), (0,)), ((), ())),
                           preferred_element_type=jnp.float32, precision=lax.Precision.HIGHEST) + offr_ref[...]
    idx_ref[...] = jnp.floor(idxf + 0.5).astype(jnp.int32)


def _run_search(xq, Wq3, bq3, kh3):
    return pl.pallas_call(
        _search_body,
        grid=(HEADS, NQB),
        in_specs=[
            pl.BlockSpec((QB, DIM), lambda h, qb: (qb, 0)),
            pl.BlockSpec((1, EMBED, DIM), lambda h, qb: (h, 0, 0)),
            pl.BlockSpec((1, 1, EMBED), lambda h, qb: (h, 0, 0)),
            pl.BlockSpec((1, N, EMBED), lambda h, qb: (h, 0, 0)),
            pl.BlockSpec((KNN, IDXW), lambda h, qb: (0, 0)),
            pl.BlockSpec((1, IDXW), lambda h, qb: (0, 0)),
            pl.BlockSpec((KNN, KNN * 16), lambda h, qb: (0, 0)),
        ],
        out_specs=[
            pl.BlockSpec((QB, IDXW), lambda h, qb: (h * NQB + qb, 0)),
            pl.BlockSpec((QB, KNN * 16), lambda h, qb: (h * NQB + qb, 0)),
        ],
        out_shape=[
            jax.ShapeDtypeStruct((ITEMS, IDXW), jnp.int32),
            jax.ShapeDtypeStruct((ITEMS, KNN * 16), jnp.float32),
        ],
    )(xq, Wq3, bq3, kh3, jnp.asarray(_S256), jnp.asarray(_OFFROW),
      jnp.asarray(_E160))


def _sc_body(idx_hbm, pr_hbm, tab_hbm, out_hbm, idx_v, pv, rows_v, acc_v, sem):
    wid = lax.axis_index("s") * NC + lax.axis_index("c")

    def item_body(i, carry):
        item = wid * PER_W + i
        pltpu.sync_copy(idx_hbm.at[item], idx_v)
        pltpu.sync_copy(pr_hbm.at[item], pv)

        def z_body(z, cz):
            acc_v[pl.ds(z * 16, 16)] = jnp.zeros((16,), jnp.float32)
            return cz

        lax.fori_loop(0, PS2 * EMBED // 16, z_body, 0)

        for c in range(2):
            pltpu.async_copy(tab_hbm.at[idx_v.at[c]], rows_v, sem).wait()
            for k in range(KNN):
                lo = max(k * PS2, c * 128) - c * 128
                hi = min((k + 1) * PS2, 250, (c + 1) * 128) - c * 128
                if lo >= hi:
                    continue
                pk = pv[k, :]
                poff = (c * 128 - k * PS2) * EMBED

                def l_body(l, cl, pk=pk, poff=poff):
                    r0 = rows_v[l, pl.ds(0, 16)]
                    r1 = rows_v[l, pl.ds(16, 16)]
                    o = l * EMBED + poff
                    acc_v[pl.ds(o, 16)] = acc_v[pl.ds(o, 16)] + pk * r0
                    acc_v[pl.ds(o + 16, 16)] = (
                        acc_v[pl.ds(o + 16, 16)] + pk * r1)
                    return cl

                lax.fori_loop(lo, hi, l_body, 0)
        pltpu.sync_copy(acc_v, out_hbm.at[item])
        return carry

    lax.fori_loop(0, PER_W, item_body, 0)


def _sc_agg(idx3, probs3, vtab):
    mesh = plsc.VectorSubcoreMesh(core_axis_name="c", subcore_axis_name="s")
    run = functools.partial(
        pl.kernel,
        out_type=jax.ShapeDtypeStruct((ITEMS, PS2 * EMBED), jnp.float32),
        mesh=mesh,
        scratch_types=[
            pltpu.VMEM((2, 128), jnp.int32),
            pltpu.VMEM((KNN, 16), jnp.float32),
            pltpu.VMEM((128, 128), jnp.float32),
            pltpu.VMEM((PS2 * EMBED,), jnp.float32),
            pltpu.SemaphoreType.DMA,
        ],
    )(_sc_body)
    return run(idx3, probs3, vtab)


def _fold_body(p_ref, wp_ref, bp_ref, ci_ref, cm_ref, out_ref):
    h = pl.program_id(0)
    if True:
        ph = p_ref[...].reshape(T, NG, NG, PS, PS, EMBED)
        a = jnp.transpose(ph, (0, 1, 3, 2, 4, 5))
        main_h = a[:, :, 0:4]
        shift_h = a[:, 0:NG - 1, 4:5]
        col0 = jnp.concatenate(
            [jnp.zeros((T, 1, 1, NG, PS, EMBED), jnp.float32), shift_h], axis=1)
        dh = jnp.concatenate(
            [col0, jnp.zeros((T, NG, 3, NG, PS, EMBED), jnp.float32)], axis=2)
        sum_h = (main_h + dh).reshape(T, H, NG, PS, EMBED)
        p64h = a[:, NG - 1:NG, 4:5].reshape(T, 1, NG, PS, EMBED)
        full_h = jnp.concatenate(
            [sum_h[:, 2:H], p64h,
             jnp.zeros((T, 1, NG, PS, EMBED), jnp.float32)], axis=1)
        main_w = full_h[:, :, :, 0:4]
        shift_w = full_h[:, :, 0:NG - 1, 4:5]
        wcol0 = jnp.concatenate(
            [jnp.zeros((T, H, 1, 1, EMBED), jnp.float32), shift_w], axis=2)
        dw = jnp.concatenate(
            [wcol0, jnp.zeros((T, H, NG, 3, EMBED), jnp.float32)], axis=3)
        sum_w = (main_w + dw).reshape(T, H, W, EMBED)
        p64w = full_h[:, :, NG - 1:NG, 4:5].reshape(T, H, 1, EMBED)
        full = jnp.concatenate(
            [sum_w[:, :, 2:W], p64w,
             jnp.zeros((T, H, 1, EMBED), jnp.float32)], axis=2)
        flat = full.reshape(N, EMBED)
        contrib = lax.dot_general(flat, wp_ref[0], (((1,), (1,)), ((), ())),
                                  preferred_element_type=jnp.float32)

        @pl.when(h == 0)
        def _():
            out_ref[...] = contrib

        @pl.when(h != 0)
        def _():
            out_ref[...] = out_ref[...] + contrib

        @pl.when(h == HEADS - 1)
        def _():
            out_ref[...] = (out_ref[...] * ci_ref[...] +
                            bp_ref[...] * cm_ref[...])


def _run_fold(patches, Wp3, bp, cinv_flat, cmask_flat):
    return pl.pallas_call(
        _fold_body,
        grid=(HEADS,),
        in_specs=[
            pl.BlockSpec((Q, PS2 * EMBED), lambda h: (h, 0)),
            pl.BlockSpec((1, DIM, EMBED), lambda h: (h, 0, 0)),
            pl.BlockSpec((1, DIM), lambda h: (0, 0)),
            pl.BlockSpec((N, 1), lambda h: (0, 0)),
            pl.BlockSpec((N, 1), lambda h: (0, 0)),
        ],
        out_specs=pl.BlockSpec((N, DIM), lambda h: (0, 0)),
        out_shape=jax.ShapeDtypeStruct((N, DIM), jnp.float32),
    )(patches, Wp3, bp.reshape(1, DIM), cinv_flat, cmask_flat)


def kernel(vid, Wq, bq, Wk, bk, Wv, bv, Wp, bp):
    x = jnp.transpose(vid, (0, 1, 3, 4, 2)).reshape(N, DIM)
    kfeat, vfeat = _run_proj(x, Wk, bk, Wv, bv)

    xq = x.reshape(T, H, W, DIM)[:, ::STRIDE, ::STRIDE, :].reshape(Q, DIM)
    kh3 = kfeat.reshape(N, HEADS, EMBED).transpose(1, 0, 2)
    Wq3 = Wq.reshape(HEADS, EMBED, DIM)
    bq3 = bq.reshape(HEADS, 1, EMBED)
    idx, probs = _run_search(xq, Wq3, bq3, kh3)

    vtab = jnp.pad(
        vfeat.reshape(T, H, W, HEADS, EMBED).transpose(3, 0, 1, 2, 4),
        ((0, 0), (0, 0), (PADW, PADW), (PADW, PADW), (0, EMBED * 3)),
    ).reshape(VROWS, DIM)
    patches = _sc_agg(idx.reshape(ITEMS, 2, 128),
                      probs.reshape(ITEMS, KNN, 16), vtab)

    Wp3 = Wp.reshape(DIM, HEADS, EMBED).transpose(1, 0, 2)
    cinv = jnp.asarray(np.broadcast_to(_CINV[None], (T, H, W)).reshape(N, 1))
    cmask = jnp.asarray(np.broadcast_to(_CMASK[None], (T, H, W)).reshape(N, 1))
    out = _run_fold(patches, Wp3, bp, cinv, cmask)
    return out.reshape(1, T, H, W, DIM).transpose(0, 1, 4, 2, 3)


# speedup vs baseline: 16.4404x; 1.0024x over previous
import functools

import numpy as np
import jax
import jax.numpy as jnp
from jax import lax
from jax.experimental import pallas as pl
from jax.experimental.pallas import tpu as pltpu
from jax.experimental.pallas import tpu_sc as plsc

EMBED = 32
HEADS = 4
DIM = 128
PS = 5
PS2 = PS * PS
KNN = 10
STRIDE = 4
PADW = 2
T = 2
H = 64
W = 64
N = T * H * W
NG = 16
Q = T * NG * NG
HP = H + 2 * PADW
VROWS = HEADS * T * HP * HP
ITEMS = HEADS * Q
QB = 128
NQB = Q // QB
NC, NS = 2, 16
NW = NC * NS
PER_W = ITEMS // NW
IDXW = 256

_OFF = (np.arange(PS)[:, None] * HP + np.arange(PS)[None, :]).reshape(-1)
_S256 = np.zeros((KNN, IDXW), np.float32)
for _k in range(KNN):
    _S256[_k, _k * PS2:(_k + 1) * PS2] = 1.0
_OFFROW = np.zeros((1, IDXW), np.float32)
_OFFROW[0, :KNN * PS2] = np.tile(_OFF.astype(np.float32), KNN)
_E160 = np.zeros((KNN, KNN * 16), np.float32)
for _k in range(KNN):
    _E160[_k, _k * 16:(_k + 1) * 16] = 1.0

_c1 = np.zeros(HP, np.float32)
for _m in range(NG):
    for _oi in range(PS):
        _c1[_m * STRIDE + _oi] += 1.0
_c1 = _c1[PADW:PADW + H]
_CNT = _c1[:, None] * _c1[None, :]
_CINV = (1.0 / np.maximum(_CNT, 1.0)).astype(np.float32)
_CMASK = (_CNT > 0).astype(np.float32)


def _proj_body(x_ref, wk_ref, bk_ref, wv_ref, bv_ref, k_ref, v_ref):
    xb = x_ref[...]
    dn = (((1,), (1,)), ((), ()))
    k_ref[...] = lax.dot_general(xb, wk_ref[...], dn,
                                 preferred_element_type=jnp.float32) + bk_ref[...]
    v_ref[...] = lax.dot_general(xb, wv_ref[...], dn,
                                 preferred_element_type=jnp.float32) + bv_ref[...]


def _run_proj(x, Wk, bk, Wv, bv):
    nblk = N // 512
    return pl.pallas_call(
        _proj_body,
        grid=(nblk,),
        in_specs=[
            pl.BlockSpec((512, DIM), lambda i: (i, 0)),
            pl.BlockSpec((DIM, DIM), lambda i: (0, 0)),
            pl.BlockSpec((1, DIM), lambda i: (0, 0)),
            pl.BlockSpec((DIM, DIM), lambda i: (0, 0)),
            pl.BlockSpec((1, DIM), lambda i: (0, 0)),
        ],
        out_specs=[
            pl.BlockSpec((512, DIM), lambda i: (i, 0)),
            pl.BlockSpec((512, DIM), lambda i: (i, 0)),
        ],
        out_shape=[
            jax.ShapeDtypeStruct((N, DIM), jnp.float32),
            jax.ShapeDtypeStruct((N, DIM), jnp.float32),
        ],
    )(x, Wk, bk.reshape(1, DIM), Wv, bv.reshape(1, DIM))


def _search_body(xq_ref, wq_ref, bq_ref, kh_ref, s256_ref, offr_ref, e160_ref,
                 idx_ref, probs_ref):
    h = pl.program_id(0)
    dn = (((1,), (1,)), ((), ()))
    xb = xq_ref[...]
    qh = lax.dot_general(xb, wq_ref[0], dn,
                         preferred_element_type=jnp.float32) + bq_ref[0]
    kh = kh_ref[0]
    kn = jnp.sum(kh * kh, axis=1)
    qn = jnp.sum(qh * qh, axis=1)
    qk = lax.dot_general(qh, kh, dn, preferred_element_type=jnp.float32)
    s = (2.0 * qk - qn[:, None]) - kn[None, :]
    iota = lax.broadcasted_iota(jnp.int32, (QB, N), 1)
    vals, inds = [], []
    for _ in range(KNN):
        m = jnp.max(s, axis=1, keepdims=True)
        a = jnp.argmax(s, axis=1).astype(jnp.int32)[:, None]
        vals.append(m)
        inds.append(a)
        s = jnp.where(iota == a, -jnp.inf, s)
    v10 = jnp.concatenate(vals, axis=1)
    i10 = jnp.concatenate(inds, axis=1)
    e = jnp.exp(v10 - v10[:, 0:1])
    p = e / jnp.sum(e, axis=1, keepdims=True)
    probs_ref[...] = lax.dot_general(p, e160_ref[...], (((1,), (0,)), ((), ())),
                                     preferred_element_type=jnp.float32, precision=lax.Precision.HIGHEST)
    t = i10 >> 12
    rem = i10 & 4095
    y = rem >> 6
    xw = rem & 63
    base = ((h * T + t) * HP + y) * HP + xw
    basef = base.astype(jnp.float32)
    idxf = lax.dot_general(basef, s256_ref[...], (((1,), (0,)), ((), ())),
                           preferred_element_type=jnp.float32, precision=lax.Precision.HIGHEST) + offr_ref[...]
    idx_ref[...] = jnp.floor(idxf + 0.5).astype(jnp.int32)


def _run_search(xq, Wq3, bq3, kh3):
    return pl.pallas_call(
        _search_body,
        grid=(HEADS, NQB),
        in_specs=[
            pl.BlockSpec((QB, DIM), lambda h, qb: (qb, 0)),
            pl.BlockSpec((1, EMBED, DIM), lambda h, qb: (h, 0, 0)),
            pl.BlockSpec((1, 1, EMBED), lambda h, qb: (h, 0, 0)),
            pl.BlockSpec((1, N, EMBED), lambda h, qb: (h, 0, 0)),
            pl.BlockSpec((KNN, IDXW), lambda h, qb: (0, 0)),
            pl.BlockSpec((1, IDXW), lambda h, qb: (0, 0)),
            pl.BlockSpec((KNN, KNN * 16), lambda h, qb: (0, 0)),
        ],
        out_specs=[
            pl.BlockSpec((QB, IDXW), lambda h, qb: (h * NQB + qb, 0)),
            pl.BlockSpec((QB, KNN * 16), lambda h, qb: (h * NQB + qb, 0)),
        ],
        out_shape=[
            jax.ShapeDtypeStruct((ITEMS, IDXW), jnp.int32),
            jax.ShapeDtypeStruct((ITEMS, KNN * 16), jnp.float32),
        ],
    )(xq, Wq3, bq3, kh3, jnp.asarray(_S256), jnp.asarray(_OFFROW),
      jnp.asarray(_E160))


def _sc_body(idx_hbm, pr_hbm, tab_hbm, out_hbm, idx_v, pv, rows_v, acc_v, sem):
    wid = lax.axis_index("s") * NC + lax.axis_index("c")

    def item_body(i, carry):
        item = wid * PER_W + i
        pltpu.sync_copy(idx_hbm.at[item], idx_v)
        pltpu.sync_copy(pr_hbm.at[item], pv)
        cp0 = pltpu.async_copy(tab_hbm.at[idx_v.at[0]],
                               rows_v.at[pl.ds(0, 128)], sem)
        cp1 = pltpu.async_copy(tab_hbm.at[idx_v.at[1]],
                               rows_v.at[pl.ds(128, 128)], sem)
        cp0.wait()
        cp1.wait()

        def p_body(p, cc):
            acc0 = jnp.zeros((16,), jnp.float32)
            acc1 = jnp.zeros((16,), jnp.float32)
            for k in range(KNN):
                pk = pv[k, :]
                j = k * PS2 + p
                acc0 = acc0 + pk * rows_v[j, pl.ds(0, 16)]
                acc1 = acc1 + pk * rows_v[j, pl.ds(16, 16)]
            acc_v[pl.ds(p * EMBED, 16)] = acc0
            acc_v[pl.ds(p * EMBED + 16, 16)] = acc1
            return cc

        lax.fori_loop(0, PS2, p_body, 0)
        pltpu.sync_copy(acc_v, out_hbm.at[item])
        return carry

    lax.fori_loop(0, PER_W, item_body, 0)


def _sc_agg(idx3, probs3, vtab):
    mesh = plsc.VectorSubcoreMesh(core_axis_name="c", subcore_axis_name="s")
    run = functools.partial(
        pl.kernel,
        out_type=jax.ShapeDtypeStruct((ITEMS, PS2 * EMBED), jnp.float32),
        mesh=mesh,
        scratch_types=[
            pltpu.VMEM((2, 128), jnp.int32),
            pltpu.VMEM((KNN, 16), jnp.float32),
            pltpu.VMEM((IDXW, 128), jnp.float32),
            pltpu.VMEM((PS2 * EMBED,), jnp.float32),
            pltpu.SemaphoreType.DMA,
        ],
    )(_sc_body)
    return run(idx3, probs3, vtab)


def _fold_body(p_ref, wp_ref, bp_ref, ci_ref, cm_ref, out_ref):
    h = pl.program_id(0)
    if True:
        ph = p_ref[...].reshape(T, NG, NG, PS, PS, EMBED)
        a = jnp.transpose(ph, (0, 1, 3, 2, 4, 5))
        main_h = a[:, :, 0:4]
        shift_h = a[:, 0:NG - 1, 4:5]
        col0 = jnp.concatenate(
            [jnp.zeros((T, 1, 1, NG, PS, EMBED), jnp.float32), shift_h], axis=1)
        dh = jnp.concatenate(
            [col0, jnp.zeros((T, NG, 3, NG, PS, EMBED), jnp.float32)], axis=2)
        sum_h = (main_h + dh).reshape(T, H, NG, PS, EMBED)
        p64h = a[:, NG - 1:NG, 4:5].reshape(T, 1, NG, PS, EMBED)
        full_h = jnp.concatenate(
            [sum_h[:, 2:H], p64h,
             jnp.zeros((T, 1, NG, PS, EMBED), jnp.float32)], axis=1)
        main_w = full_h[:, :, :, 0:4]
        shift_w = full_h[:, :, 0:NG - 1, 4:5]
        wcol0 = jnp.concatenate(
            [jnp.zeros((T, H, 1, 1, EMBED), jnp.float32), shift_w], axis=2)
        dw = jnp.concatenate(
            [wcol0, jnp.zeros((T, H, NG, 3, EMBED), jnp.float32)], axis=3)
        sum_w = (main_w + dw).reshape(T, H, W, EMBED)
        p64w = full_h[:, :, NG - 1:NG, 4:5].reshape(T, H, 1, EMBED)
        full = jnp.concatenate(
            [sum_w[:, :, 2:W], p64w,
             jnp.zeros((T, H, 1, EMBED), jnp.float32)], axis=2)
        flat = full.reshape(N, EMBED)
        contrib = lax.dot_general(flat, wp_ref[0], (((1,), (1,)), ((), ())),
                                  preferred_element_type=jnp.float32)

        @pl.when(h == 0)
        def _():
            out_ref[...] = contrib

        @pl.when(h != 0)
        def _():
            out_ref[...] = out_ref[...] + contrib

        @pl.when(h == HEADS - 1)
        def _():
            out_ref[...] = (out_ref[...] * ci_ref[...] +
                            bp_ref[...] * cm_ref[...])


def _run_fold(patches, Wp3, bp, cinv_flat, cmask_flat):
    return pl.pallas_call(
        _fold_body,
        grid=(HEADS,),
        in_specs=[
            pl.BlockSpec((Q, PS2 * EMBED), lambda h: (h, 0)),
            pl.BlockSpec((1, DIM, EMBED), lambda h: (h, 0, 0)),
            pl.BlockSpec((1, DIM), lambda h: (0, 0)),
            pl.BlockSpec((N, 1), lambda h: (0, 0)),
            pl.BlockSpec((N, 1), lambda h: (0, 0)),
        ],
        out_specs=pl.BlockSpec((N, DIM), lambda h: (0, 0)),
        out_shape=jax.ShapeDtypeStruct((N, DIM), jnp.float32),
    )(patches, Wp3, bp.reshape(1, DIM), cinv_flat, cmask_flat)


def kernel(vid, Wq, bq, Wk, bk, Wv, bv, Wp, bp):
    x = jnp.transpose(vid, (0, 1, 3, 4, 2)).reshape(N, DIM)
    kfeat, vfeat = _run_proj(x, Wk, bk, Wv, bv)

    xq = x.reshape(T, H, W, DIM)[:, ::STRIDE, ::STRIDE, :].reshape(Q, DIM)
    kh3 = kfeat.reshape(N, HEADS, EMBED).transpose(1, 0, 2)
    Wq3 = Wq.reshape(HEADS, EMBED, DIM)
    bq3 = bq.reshape(HEADS, 1, EMBED)
    idx, probs = _run_search(xq, Wq3, bq3, kh3)

    vtab = jnp.pad(
        vfeat.reshape(T, H, W, HEADS, EMBED).transpose(3, 0, 1, 2, 4),
        ((0, 0), (0, 0), (PADW, PADW), (PADW, PADW), (0, EMBED * 3)),
    ).reshape(VROWS, DIM)
    patches = _sc_agg(idx.reshape(ITEMS, 2, 128),
                      probs.reshape(ITEMS, KNN, 16), vtab)

    Wp3 = Wp.reshape(DIM, HEADS, EMBED).transpose(1, 0, 2)
    cinv = jnp.asarray(np.broadcast_to(_CINV[None], (T, H, W)).reshape(N, 1))
    cmask = jnp.asarray(np.broadcast_to(_CMASK[None], (T, H, W)).reshape(N, 1))
    out = _run_fold(patches, Wp3, bp, cinv, cmask)
    return out.reshape(1, T, H, W, DIM).transpose(0, 1, 4, 2, 3)
